# Initial kernel scaffold; baseline (speedup 1.0000x reference)
import jax
import jax.numpy as jnp
from jax.experimental import pallas as pl


def kernel(x, edge_index, edge_attr, params):
    raise NotImplementedError("write your pallas kernel here")


# speedup vs baseline: 2.8635x; 2.8635x over previous
import functools

import jax
import jax.numpy as jnp
from jax import lax
from jax.experimental import pallas as pl
from jax.experimental.pallas import tpu as pltpu
from jax.experimental.pallas import tpu_sc as plsc

N_NODES = 10000
N_EDGES = 320000
D = 128
D_EDGE = 16
N_LAYERS = 3

GP = 128
NGROUPS = N_EDGES // GP
NC, NS = 2, 16
NW = NC * NS
GROUPS_PER_TILE = -(-NGROUPS // NW)
AGG_ROWS = 10240
ROWS_PER_TILE = AGG_ROWS // NS

BE = 2000
BN = 2000


def _edge_proj_body(ea_ref, w_ref, b_ref, o0_ref, o1_ref, o2_ref):
    y = jnp.dot(ea_ref[...], w_ref[...], preferred_element_type=jnp.float32)
    y = y + b_ref[...]
    o0_ref[...] = y[:, 0 * D:1 * D]
    o1_ref[...] = y[:, 1 * D:2 * D]
    o2_ref[...] = y[:, 2 * D:3 * D]


def _edge_proj(edge_attr, w_cat, b_cat):
    return pl.pallas_call(
        _edge_proj_body,
        grid=(N_EDGES // BE,),
        in_specs=[
            pl.BlockSpec((BE, D_EDGE), lambda i: (i, 0)),
            pl.BlockSpec((D_EDGE, N_LAYERS * D), lambda i: (0, 0)),
            pl.BlockSpec((1, N_LAYERS * D), lambda i: (0, 0)),
        ],
        out_specs=[pl.BlockSpec((BE, D), lambda i: (i, 0))] * N_LAYERS,
        out_shape=[jax.ShapeDtypeStruct((N_EDGES, D), jnp.float32)] * N_LAYERS,
    )(edge_attr, w_cat, b_cat)


def _node_update_body(h_ref, agg_ref, w1_ref, b1_ref, w2_ref, b2_ref,
                      g_ref, bb_ref, o_ref):
    u = h_ref[...] + agg_ref[0] + agg_ref[1]
    z = jnp.dot(u, w1_ref[...], preferred_element_type=jnp.float32) + b1_ref[...]
    z = jnp.maximum(z, 0.0)
    z = jnp.dot(z, w2_ref[...], preferred_element_type=jnp.float32) + b2_ref[...]
    mu = jnp.mean(z, axis=-1, keepdims=True)
    zc = z - mu
    var = jnp.mean(zc * zc, axis=-1, keepdims=True)
    o_ref[...] = zc * lax.rsqrt(var + 1e-5) * g_ref[...] + bb_ref[...]


def _node_update(h, agg, lp):
    return pl.pallas_call(
        _node_update_body,
        grid=(N_NODES // BN,),
        in_specs=[
            pl.BlockSpec((BN, D), lambda i: (i, 0)),
            pl.BlockSpec((NC, BN, D), lambda i: (0, i, 0)),
            pl.BlockSpec((D, D), lambda i: (0, 0)),
            pl.BlockSpec((1, D), lambda i: (0, 0)),
            pl.BlockSpec((D, D), lambda i: (0, 0)),
            pl.BlockSpec((1, D), lambda i: (0, 0)),
            pl.BlockSpec((1, D), lambda i: (0, 0)),
            pl.BlockSpec((1, D), lambda i: (0, 0)),
        ],
        out_specs=pl.BlockSpec((BN, D), lambda i: (i, 0)),
        out_shape=jax.ShapeDtypeStruct((N_NODES, D), jnp.float32),
    )(h, agg, lp["W1"], lp["b1"].reshape(1, D), lp["W2"],
      lp["b2"].reshape(1, D), lp["ln_g"].reshape(1, D), lp["ln_b"].reshape(1, D))


def _head_body(h_ref, w1_ref, b1_ref, w2_ref, b2_ref, o_ref):
    z = jnp.dot(h_ref[...], w1_ref[...], preferred_element_type=jnp.float32)
    z = jnp.maximum(z + b1_ref[...], 0.0)
    o = jnp.dot(z, w2_ref[...], preferred_element_type=jnp.float32) + b2_ref[...]
    o_ref[...] = jax.nn.sigmoid(o)


def _head(h, hp):
    return pl.pallas_call(
        _head_body,
        grid=(N_NODES // BN,),
        in_specs=[
            pl.BlockSpec((BN, D), lambda i: (i, 0)),
            pl.BlockSpec((D, D), lambda i: (0, 0)),
            pl.BlockSpec((1, D), lambda i: (0, 0)),
            pl.BlockSpec((D, 1), lambda i: (0, 0)),
            pl.BlockSpec((1, 1), lambda i: (0, 0)),
        ],
        out_specs=pl.BlockSpec((BN, 1), lambda i: (i, 0)),
        out_shape=jax.ShapeDtypeStruct((N_NODES, 1), jnp.float32),
    )(h, hp["W1"], hp["b1"].reshape(1, D), hp["W2"], hp["b2"].reshape(1, 1))


@functools.partial(
    pl.kernel,
    out_type=jax.ShapeDtypeStruct((NC, AGG_ROWS, D), jnp.float32),
    mesh=plsc.VectorSubcoreMesh(core_axis_name="c", subcore_axis_name="s"),
    scratch_types=[
        pltpu.VMEM((GP,), jnp.int32),
        pltpu.VMEM((GP,), jnp.int32),
        pltpu.VMEM((GP, D), jnp.float32),
        pltpu.VMEM((GP, D), jnp.float32),
        pltpu.VMEM_SHARED((AGG_ROWS, D), jnp.float32),
        pltpu.SemaphoreType.DMA,
    ],
)
def _sc_aggregate(h_hbm, e_hbm, srcg_hbm, dstg_hbm, out_hbm,
                  src_v, dst_v, rows_v, e_v, agg_sh, sem):
    c = lax.axis_index("c")
    s = lax.axis_index("s")
    wid = s * NC + c

    zvec = jnp.zeros((16,), jnp.float32)

    def zbody(i, carry):
        for j in range(D // 16):
            rows_v[i, pl.ds(j * 16, 16)] = zvec
        return carry

    lax.fori_loop(0, GP, zbody, 0)
    for k in range(ROWS_PER_TILE // GP):
        pltpu.sync_copy(
            rows_v, agg_sh.at[pl.ds(s * ROWS_PER_TILE + k * GP, GP), :])
    plsc.subcore_barrier()

    def body(it, carry):
        g = it * NW + wid

        @pl.when(g < NGROUPS)
        def _():
            pltpu.sync_copy(srcg_hbm.at[pl.ds(g * GP, GP)], src_v)
            pltpu.sync_copy(dstg_hbm.at[pl.ds(g * GP, GP)], dst_v)
            pltpu.async_copy(h_hbm.at[src_v], rows_v, sem).wait()
            pltpu.sync_copy(e_hbm.at[pl.ds(g * GP, GP), :], e_v)

            def vbody(i, carry2):
                for j in range(D // 16):
                    sl = pl.ds(j * 16, 16)
                    rows_v[i, sl] = jnp.maximum(rows_v[i, sl] + e_v[i, sl], 0.0)
                return carry2

            lax.fori_loop(0, GP, vbody, 0)
            pltpu.sync_copy(rows_v, agg_sh.at[dst_v], add=True)

        return carry

    lax.fori_loop(0, GROUPS_PER_TILE, body, 0)
    plsc.subcore_barrier()

    row0 = s * ROWS_PER_TILE
    pltpu.sync_copy(agg_sh.at[pl.ds(row0, ROWS_PER_TILE), :],
                    out_hbm.at[c, pl.ds(row0, ROWS_PER_TILE), :])


def kernel(x, edge_index, edge_attr, params):
    srcg = edge_index[0].astype(jnp.int32)
    dstg = edge_index[1].astype(jnp.int32)

    w_cat = jnp.concatenate([lp["We"] for lp in params["layers"]], axis=1)
    b_cat = jnp.concatenate([lp["be"] for lp in params["layers"]]).reshape(1, -1)
    eprojs = _edge_proj(edge_attr, w_cat, b_cat)

    h = x
    for l, lp in enumerate(params["layers"]):
        agg = _sc_aggregate(h, eprojs[l], srcg, dstg)
        h = _node_update(h, agg, lp)
    return _head(h, params["head"])


# speedup vs baseline: 4.3706x; 1.5263x over previous
import functools

import jax
import jax.numpy as jnp
from jax import lax
from jax.experimental import pallas as pl
from jax.experimental.pallas import tpu as pltpu
from jax.experimental.pallas import tpu_sc as plsc

N_NODES = 10000
N_EDGES = 320000
D = 128
D_EDGE = 16
N_LAYERS = 3

GP = 64
NGROUPS = N_EDGES // GP
NC, NS = 2, 16
NW = NC * NS
GROUPS_PER_TILE = -(-NGROUPS // NW)
AGG_ROWS = 10240
ROWS_PER_TILE = AGG_ROWS // NS

BE = 2000
BN = 2000


def _edge_proj_body(ea_ref, w_ref, b_ref, o0_ref, o1_ref, o2_ref):
    y = jnp.dot(ea_ref[...], w_ref[...], preferred_element_type=jnp.float32)
    y = y + b_ref[...]
    o0_ref[...] = y[:, 0 * D:1 * D]
    o1_ref[...] = y[:, 1 * D:2 * D]
    o2_ref[...] = y[:, 2 * D:3 * D]


def _edge_proj(edge_attr, w_cat, b_cat):
    return pl.pallas_call(
        _edge_proj_body,
        grid=(N_EDGES // BE,),
        in_specs=[
            pl.BlockSpec((BE, D_EDGE), lambda i: (i, 0)),
            pl.BlockSpec((D_EDGE, N_LAYERS * D), lambda i: (0, 0)),
            pl.BlockSpec((1, N_LAYERS * D), lambda i: (0, 0)),
        ],
        out_specs=[pl.BlockSpec((BE, D), lambda i: (i, 0))] * N_LAYERS,
        out_shape=[jax.ShapeDtypeStruct((N_EDGES, D), jnp.float32)] * N_LAYERS,
    )(edge_attr, w_cat, b_cat)


def _node_update_body(h_ref, agg_ref, w1_ref, b1_ref, w2_ref, b2_ref,
                      g_ref, bb_ref, o_ref):
    u = h_ref[...] + agg_ref[0] + agg_ref[1]
    z = jnp.dot(u, w1_ref[...], preferred_element_type=jnp.float32) + b1_ref[...]
    z = jnp.maximum(z, 0.0)
    z = jnp.dot(z, w2_ref[...], preferred_element_type=jnp.float32) + b2_ref[...]
    mu = jnp.mean(z, axis=-1, keepdims=True)
    zc = z - mu
    var = jnp.mean(zc * zc, axis=-1, keepdims=True)
    o_ref[...] = zc * lax.rsqrt(var + 1e-5) * g_ref[...] + bb_ref[...]


def _node_update(h, agg, lp):
    return pl.pallas_call(
        _node_update_body,
        grid=(N_NODES // BN,),
        in_specs=[
            pl.BlockSpec((BN, D), lambda i: (i, 0)),
            pl.BlockSpec((NC, BN, D), lambda i: (0, i, 0)),
            pl.BlockSpec((D, D), lambda i: (0, 0)),
            pl.BlockSpec((1, D), lambda i: (0, 0)),
            pl.BlockSpec((D, D), lambda i: (0, 0)),
            pl.BlockSpec((1, D), lambda i: (0, 0)),
            pl.BlockSpec((1, D), lambda i: (0, 0)),
            pl.BlockSpec((1, D), lambda i: (0, 0)),
        ],
        out_specs=pl.BlockSpec((BN, D), lambda i: (i, 0)),
        out_shape=jax.ShapeDtypeStruct((N_NODES, D), jnp.float32),
    )(h, agg, lp["W1"], lp["b1"].reshape(1, D), lp["W2"],
      lp["b2"].reshape(1, D), lp["ln_g"].reshape(1, D), lp["ln_b"].reshape(1, D))


def _head_body(h_ref, w1_ref, b1_ref, w2_ref, b2_ref, o_ref):
    z = jnp.dot(h_ref[...], w1_ref[...], preferred_element_type=jnp.float32)
    z = jnp.maximum(z + b1_ref[...], 0.0)
    o = jnp.dot(z, w2_ref[...], preferred_element_type=jnp.float32) + b2_ref[...]
    o_ref[...] = jax.nn.sigmoid(o)


def _head(h, hp):
    return pl.pallas_call(
        _head_body,
        grid=(N_NODES // BN,),
        in_specs=[
            pl.BlockSpec((BN, D), lambda i: (i, 0)),
            pl.BlockSpec((D, D), lambda i: (0, 0)),
            pl.BlockSpec((1, D), lambda i: (0, 0)),
            pl.BlockSpec((D, 1), lambda i: (0, 0)),
            pl.BlockSpec((1, 1), lambda i: (0, 0)),
        ],
        out_specs=pl.BlockSpec((BN, 1), lambda i: (i, 0)),
        out_shape=jax.ShapeDtypeStruct((N_NODES, 1), jnp.float32),
    )(h, hp["W1"], hp["b1"].reshape(1, D), hp["W2"], hp["b2"].reshape(1, 1))


@functools.partial(
    pl.kernel,
    out_type=jax.ShapeDtypeStruct((NC, AGG_ROWS, D), jnp.float32),
    mesh=plsc.VectorSubcoreMesh(core_axis_name="c", subcore_axis_name="s"),
    scratch_types=[
        pltpu.VMEM((2, GP), jnp.int32),
        pltpu.VMEM((2, GP), jnp.int32),
        pltpu.VMEM((2, GP, D), jnp.float32),
        pltpu.VMEM((2, GP, D), jnp.float32),
        pltpu.VMEM_SHARED((AGG_ROWS, D), jnp.float32),
        pltpu.SemaphoreType.DMA,
        pltpu.SemaphoreType.DMA,
        pltpu.SemaphoreType.DMA,
        pltpu.SemaphoreType.DMA,
    ],
)
def _sc_aggregate(h_hbm, e_hbm, srcg_hbm, dstg_hbm, out_hbm,
                  src_v, dst_v, rows_v, e_v, agg_sh,
                  sem_g0, sem_g1, sem_i0, sem_i1):
    c = lax.axis_index("c")
    s = lax.axis_index("s")
    wid = s * NC + c
    sem_g = (sem_g0, sem_g1)
    sem_i = (sem_i0, sem_i1)

    def issue_idx(g, p):
        pltpu.async_copy(srcg_hbm.at[pl.ds(g * GP, GP)], src_v.at[p], sem_i[p])
        pltpu.async_copy(dstg_hbm.at[pl.ds(g * GP, GP)], dst_v.at[p], sem_i[p])

    def wait_idx(p):
        pltpu.make_async_copy(srcg_hbm.at[pl.ds(0, GP)], src_v.at[p],
                              sem_i[p]).wait()
        pltpu.make_async_copy(dstg_hbm.at[pl.ds(0, GP)], dst_v.at[p],
                              sem_i[p]).wait()

    def issue_rows(g, p):
        pltpu.async_copy(h_hbm.at[src_v.at[p]], rows_v.at[p], sem_g[p])
        pltpu.async_copy(e_hbm.at[pl.ds(g * GP, GP), :], e_v.at[p], sem_g[p])

    def wait_rows(p):
        pltpu.make_async_copy(h_hbm.at[pl.ds(0, GP), :], rows_v.at[p],
                              sem_g[p]).wait()
        pltpu.make_async_copy(e_hbm.at[pl.ds(0, GP), :], e_v.at[p],
                              sem_g[p]).wait()

    zvec = jnp.zeros((16,), jnp.float32)

    def zbody(i, carry):
        for j in range(D // 16):
            rows_v[0, i, pl.ds(j * 16, 16)] = zvec
        return carry

    lax.fori_loop(0, GP, zbody, 0)
    for k in range(ROWS_PER_TILE // GP):
        pltpu.sync_copy(
            rows_v.at[0], agg_sh.at[pl.ds(s * ROWS_PER_TILE + k * GP, GP), :])
    plsc.subcore_barrier()

    g0 = wid
    pltpu.sync_copy(srcg_hbm.at[pl.ds(g0 * GP, GP)], src_v.at[0])
    pltpu.sync_copy(dstg_hbm.at[pl.ds(g0 * GP, GP)], dst_v.at[0])
    issue_rows(g0, 0)

    @pl.when(g0 + NW < NGROUPS)
    def _():
        issue_idx(g0 + NW, 1)

    @pl.loop(0, GROUPS_PER_TILE + 1, step=2)
    def _k(k):
        for b in range(2):
            p = b
            q = 1 - b
            kk = k + b
            g = kk * NW + wid
            gn = g + NW
            gnn = gn + NW

            @pl.when(gn < NGROUPS)
            def _():
                wait_idx(q)
                issue_rows(gn, q)

            @pl.when(g < NGROUPS)
            def _():
                wait_rows(p)

                def vbody(i, carry2):
                    for j in range(D // 16):
                        sl = pl.ds(j * 16, 16)
                        rows_v[p, i, sl] = jnp.maximum(
                            rows_v[p, i, sl] + e_v[p, i, sl], 0.0)
                    return carry2

                lax.fori_loop(0, GP, vbody, 0)
                pltpu.sync_copy(rows_v.at[p], agg_sh.at[dst_v.at[p]], add=True)

            @pl.when(gnn < NGROUPS)
            def _():
                issue_idx(gnn, p)

    plsc.subcore_barrier()

    row0 = s * ROWS_PER_TILE
    pltpu.sync_copy(agg_sh.at[pl.ds(row0, ROWS_PER_TILE), :],
                    out_hbm.at[c, pl.ds(row0, ROWS_PER_TILE), :])


def kernel(x, edge_index, edge_attr, params):
    srcg = edge_index[0].astype(jnp.int32)
    dstg = edge_index[1].astype(jnp.int32)

    w_cat = jnp.concatenate([lp["We"] for lp in params["layers"]], axis=1)
    b_cat = jnp.concatenate([lp["be"] for lp in params["layers"]]).reshape(1, -1)
    eprojs = _edge_proj(edge_attr, w_cat, b_cat)

    h = x
    for l, lp in enumerate(params["layers"]):
        agg = _sc_aggregate(h, eprojs[l], srcg, dstg)
        h = _node_update(h, agg, lp)
    return _head(h, params["head"])
